# Initial kernel scaffold; baseline (speedup 1.0000x reference)
import functools

import jax
import jax.numpy as jnp
from jax.experimental import pallas as pl

HIDDEN = 1024
INTER = 2816
E = 8
TOPK = 2

BT = 512
IBLK = 256
BTR = 512


def _router_body(x_ref, rw_ref, rb_ref, wfull_ref):
    x = x_ref[...]
    logits = jnp.dot(x, rw_ref[...].T, preferred_element_type=jnp.float32)
    logits = logits + rb_ref[...]
    m = jnp.max(logits, axis=-1, keepdims=True)
    ex = jnp.exp(logits - m)
    probs = ex / jnp.sum(ex, axis=-1, keepdims=True)

    lane = jax.lax.broadcasted_iota(jnp.int32, (BTR, E), 1)
    m1 = jnp.max(probs, axis=-1, keepdims=True)
    a1 = jnp.min(jnp.where(probs == m1, lane, E), axis=-1, keepdims=True)
    probs2 = jnp.where(lane == a1, -1.0, probs)
    m2 = jnp.max(probs2, axis=-1, keepdims=True)
    a2 = jnp.min(jnp.where(probs2 == m2, lane, E), axis=-1, keepdims=True)

    denom = m1 + m2
    w1 = m1 / denom
    w2 = m2 / denom
    wfull = jnp.where(lane == a1, w1, 0.0) + jnp.where(lane == a2, w2, 0.0)
    out = jnp.zeros((BTR, 128), jnp.float32)
    wfull_ref[...] = out.at[:, :E].set(wfull)


def _ffn_body(wfull_ref, x_ref, gw_ref, uw_ref, dw_ref, o_ref):
    e = pl.program_id(1)
    i = pl.program_id(2)

    @pl.when((e == 0) & (i == 0))
    def _init():
        o_ref[...] = jnp.zeros_like(o_ref)

    x = x_ref[...]
    g = jnp.dot(x, gw_ref[0].T, preferred_element_type=jnp.float32)
    u = jnp.dot(x, uw_ref[0].T, preferred_element_type=jnp.float32)
    h = g * jax.lax.logistic(g) * u
    part = jnp.dot(h, dw_ref[0].T, preferred_element_type=jnp.float32)

    lane = jax.lax.broadcasted_iota(jnp.int32, (BT, 128), 1)
    w_all = wfull_ref[...]
    w = jnp.sum(jnp.where(lane == e, w_all, 0.0), axis=-1, keepdims=True)
    o_ref[...] += w * part


@jax.jit
def kernel(x, router_w, router_b, gate_w, up_w, down_w):
    batch_shape = x.shape[:-1]
    xf = x.reshape(-1, HIDDEN)
    T = xf.shape[0]

    wfull = pl.pallas_call(
        _router_body,
        grid=(T // BTR,),
        in_specs=[
            pl.BlockSpec((BTR, HIDDEN), lambda t: (t, 0)),
            pl.BlockSpec((E, HIDDEN), lambda t: (0, 0)),
            pl.BlockSpec((1, E), lambda t: (0, 0)),
        ],
        out_specs=pl.BlockSpec((BTR, 128), lambda t: (t, 0)),
        out_shape=jax.ShapeDtypeStruct((T, 128), jnp.float32),
    )(xf, router_w, router_b.reshape(1, E))

    out = pl.pallas_call(
        _ffn_body,
        grid=(T // BT, E, INTER // IBLK),
        in_specs=[
            pl.BlockSpec((BT, 128), lambda t, e, i: (t, 0)),
            pl.BlockSpec((BT, HIDDEN), lambda t, e, i: (t, 0)),
            pl.BlockSpec((1, IBLK, HIDDEN), lambda t, e, i: (e, i, 0)),
            pl.BlockSpec((1, IBLK, HIDDEN), lambda t, e, i: (e, i, 0)),
            pl.BlockSpec((1, HIDDEN, IBLK), lambda t, e, i: (e, 0, i)),
        ],
        out_specs=pl.BlockSpec((BT, HIDDEN), lambda t, e, i: (t, 0)),
        out_shape=jax.ShapeDtypeStruct((T, HIDDEN), jnp.float32),
    )(wfull, xf, gate_w, up_w, down_w)

    return out.reshape(*batch_shape, HIDDEN)


# speedup vs baseline: 1.0651x; 1.0651x over previous
import functools

import jax
import jax.numpy as jnp
from jax.experimental import pallas as pl

HIDDEN = 1024
INTER = 2816
E = 8
TOPK = 2

BT = 512
IBLK = 256
BTR = 512


def _router_body(x_ref, rw_ref, rb_ref, wfull_ref):
    x = x_ref[...]
    logits = jnp.dot(x, rw_ref[...].T, preferred_element_type=jnp.float32)
    logits = logits + rb_ref[...]
    m = jnp.max(logits, axis=-1, keepdims=True)
    ex = jnp.exp(logits - m)
    probs = ex / jnp.sum(ex, axis=-1, keepdims=True)

    lane = jax.lax.broadcasted_iota(jnp.int32, (BTR, E), 1)
    m1 = jnp.max(probs, axis=-1, keepdims=True)
    a1 = jnp.min(jnp.where(probs == m1, lane, E), axis=-1, keepdims=True)
    probs2 = jnp.where(lane == a1, -1.0, probs)
    m2 = jnp.max(probs2, axis=-1, keepdims=True)
    a2 = jnp.min(jnp.where(probs2 == m2, lane, E), axis=-1, keepdims=True)

    denom = m1 + m2
    w1 = m1 / denom
    w2 = m2 / denom
    wfull = jnp.where(lane == a1, w1, 0.0) + jnp.where(lane == a2, w2, 0.0)
    pad = jnp.zeros((BTR, 128 - E), jnp.float32)
    wfull_ref[...] = jnp.concatenate([wfull, pad], axis=-1)


def _ffn_body(wfull_ref, x_ref, gw_ref, uw_ref, dw_ref, o_ref):
    e = pl.program_id(1)
    i = pl.program_id(2)

    @pl.when((e == 0) & (i == 0))
    def _init():
        o_ref[...] = jnp.zeros_like(o_ref)

    x = x_ref[...]
    g = jnp.dot(x, gw_ref[0].T, preferred_element_type=jnp.float32)
    u = jnp.dot(x, uw_ref[0].T, preferred_element_type=jnp.float32)
    h = g * jax.lax.logistic(g) * u
    part = jnp.dot(h, dw_ref[0].T, preferred_element_type=jnp.float32)

    lane = jax.lax.broadcasted_iota(jnp.int32, (BT, 128), 1)
    w_all = wfull_ref[...]
    w = jnp.sum(jnp.where(lane == e, w_all, 0.0), axis=-1, keepdims=True)
    o_ref[...] += w * part


@jax.jit
def kernel(x, router_w, router_b, gate_w, up_w, down_w):
    batch_shape = x.shape[:-1]
    xf = x.reshape(-1, HIDDEN)
    T = xf.shape[0]

    wfull = pl.pallas_call(
        _router_body,
        grid=(T // BTR,),
        in_specs=[
            pl.BlockSpec((BTR, HIDDEN), lambda t: (t, 0)),
            pl.BlockSpec((E, HIDDEN), lambda t: (0, 0)),
            pl.BlockSpec((1, E), lambda t: (0, 0)),
        ],
        out_specs=pl.BlockSpec((BTR, 128), lambda t: (t, 0)),
        out_shape=jax.ShapeDtypeStruct((T, 128), jnp.float32),
    )(xf, router_w, router_b.reshape(1, E))

    out = pl.pallas_call(
        _ffn_body,
        grid=(T // BT, E, INTER // IBLK),
        in_specs=[
            pl.BlockSpec((BT, 128), lambda t, e, i: (t, 0)),
            pl.BlockSpec((BT, HIDDEN), lambda t, e, i: (t, 0)),
            pl.BlockSpec((1, IBLK, HIDDEN), lambda t, e, i: (e, i, 0)),
            pl.BlockSpec((1, IBLK, HIDDEN), lambda t, e, i: (e, i, 0)),
            pl.BlockSpec((1, HIDDEN, IBLK), lambda t, e, i: (e, 0, i)),
        ],
        out_specs=pl.BlockSpec((BT, HIDDEN), lambda t, e, i: (t, 0)),
        out_shape=jax.ShapeDtypeStruct((T, HIDDEN), jnp.float32),
    )(wfull, xf, gate_w, up_w, down_w)

    return out.reshape(*batch_shape, HIDDEN)


# speedup vs baseline: 1.0884x; 1.0218x over previous
import functools

import jax
import jax.numpy as jnp
from jax.experimental import pallas as pl

HIDDEN = 1024
INTER = 2816
E = 8
TOPK = 2

BT = 512
IBLK = 256
BTR = 512


def _router_body(x_ref, rw_ref, rb_ref, wfull_ref):
    x = x_ref[...]
    logits = jnp.dot(x, rw_ref[...].T, preferred_element_type=jnp.float32)
    logits = logits + rb_ref[...]
    m = jnp.max(logits, axis=-1, keepdims=True)
    ex = jnp.exp(logits - m)
    probs = ex / jnp.sum(ex, axis=-1, keepdims=True)

    lane = jax.lax.broadcasted_iota(jnp.int32, (BTR, E), 1)
    m1 = jnp.max(probs, axis=-1, keepdims=True)
    a1 = jnp.min(jnp.where(probs == m1, lane, E), axis=-1, keepdims=True)
    probs2 = jnp.where(lane == a1, -1.0, probs)
    m2 = jnp.max(probs2, axis=-1, keepdims=True)
    a2 = jnp.min(jnp.where(probs2 == m2, lane, E), axis=-1, keepdims=True)

    denom = m1 + m2
    w1 = m1 / denom
    w2 = m2 / denom
    wfull = jnp.where(lane == a1, w1, 0.0) + jnp.where(lane == a2, w2, 0.0)
    pad = jnp.zeros((BTR, 128 - E), jnp.float32)
    wfull_ref[...] = jnp.concatenate([wfull, pad], axis=-1)


def _ffn_body(wfull_ref, x_ref, gw_ref, uw_ref, dw_ref, o_ref):
    e = pl.program_id(1)
    i = pl.program_id(2)

    @pl.when((e == 0) & (i == 0))
    def _init():
        o_ref[...] = jnp.zeros_like(o_ref)

    x = x_ref[...]
    g = jnp.dot(x, gw_ref[0].T, preferred_element_type=jnp.float32)
    u = jnp.dot(x, uw_ref[0].T, preferred_element_type=jnp.float32)
    h = g * jax.lax.logistic(g) * u
    part = jnp.dot(h.astype(jnp.bfloat16), dw_ref[0].T,
                   preferred_element_type=jnp.float32)

    lane = jax.lax.broadcasted_iota(jnp.int32, (BT, 128), 1)
    w_all = wfull_ref[...]
    w = jnp.sum(jnp.where(lane == e, w_all, 0.0), axis=-1, keepdims=True)
    o_ref[...] += w * part


@jax.jit
def kernel(x, router_w, router_b, gate_w, up_w, down_w):
    batch_shape = x.shape[:-1]
    xf = x.reshape(-1, HIDDEN)
    T = xf.shape[0]

    wfull = pl.pallas_call(
        _router_body,
        grid=(T // BTR,),
        in_specs=[
            pl.BlockSpec((BTR, HIDDEN), lambda t: (t, 0)),
            pl.BlockSpec((E, HIDDEN), lambda t: (0, 0)),
            pl.BlockSpec((1, E), lambda t: (0, 0)),
        ],
        out_specs=pl.BlockSpec((BTR, 128), lambda t: (t, 0)),
        out_shape=jax.ShapeDtypeStruct((T, 128), jnp.float32),
    )(xf, router_w, router_b.reshape(1, E))

    out = pl.pallas_call(
        _ffn_body,
        grid=(T // BT, E, INTER // IBLK),
        in_specs=[
            pl.BlockSpec((BT, 128), lambda t, e, i: (t, 0)),
            pl.BlockSpec((BT, HIDDEN), lambda t, e, i: (t, 0)),
            pl.BlockSpec((1, IBLK, HIDDEN), lambda t, e, i: (e, i, 0)),
            pl.BlockSpec((1, IBLK, HIDDEN), lambda t, e, i: (e, i, 0)),
            pl.BlockSpec((1, HIDDEN, IBLK), lambda t, e, i: (e, 0, i)),
        ],
        out_specs=pl.BlockSpec((BT, HIDDEN), lambda t, e, i: (t, 0)),
        out_shape=jax.ShapeDtypeStruct((T, HIDDEN), jnp.float32),
    )(wfull, xf.astype(jnp.bfloat16), gate_w.astype(jnp.bfloat16),
      up_w.astype(jnp.bfloat16), down_w.astype(jnp.bfloat16))

    return out.reshape(*batch_shape, HIDDEN)


# speedup vs baseline: 2.6831x; 2.4653x over previous
import functools

import jax
import jax.numpy as jnp
from jax import lax
from jax.experimental import pallas as pl
from jax.experimental.pallas import tpu as pltpu
from jax.experimental.pallas import tpu_sc as plsc

HIDDEN = 1024
INTER = 2816
E = 8

T = 4096
P = 2 * T
B = 256
NBMAX = P // B + E
NPAD = NBMAX * B
BTR = 512

NW = 32
PPW = P // NW
CH = 64
TPW = T // NW
CC = 32


def _router_body(x_ref, rw_ref, rb_ref, sel_ref, w_ref):
    x = x_ref[...]
    logits = jnp.dot(x, rw_ref[...].T, preferred_element_type=jnp.float32)
    logits = logits + rb_ref[...]
    m = jnp.max(logits, axis=-1, keepdims=True)
    ex = jnp.exp(logits - m)
    probs = ex / jnp.sum(ex, axis=-1, keepdims=True)

    lane = lax.broadcasted_iota(jnp.int32, (BTR, E), 1)
    m1 = jnp.max(probs, axis=-1, keepdims=True)
    a1 = jnp.min(jnp.where(probs == m1, lane, E), axis=-1, keepdims=True)
    probs2 = jnp.where(lane == a1, -1.0, probs)
    m2 = jnp.max(probs2, axis=-1, keepdims=True)
    a2 = jnp.min(jnp.where(probs2 == m2, lane, E), axis=-1, keepdims=True)

    denom = m1 + m2
    w1 = m1 / denom
    w2 = m2 / denom
    zi = jnp.zeros((BTR, 126), jnp.int32)
    zf = jnp.zeros((BTR, 126), jnp.float32)
    sel_ref[...] = jnp.concatenate([a1, a2, zi], axis=-1)
    w_ref[...] = jnp.concatenate([w1, w2, zf], axis=-1)


def _dispatch_body(pairs_ref, pos_ref, eb_ref):
    R = pairs_ref[...]
    r0 = lax.broadcasted_iota(jnp.int32, (128, 128), 0)
    r1 = lax.broadcasted_iota(jnp.int32, (128, 128), 1)
    SU = (r0 < r1).astype(jnp.float32)
    s0 = lax.broadcasted_iota(jnp.int32, (64, 64), 0)
    s1 = lax.broadcasted_iota(jnp.int32, (64, 64), 1)
    SL = (s1 < s0).astype(jnp.float32)

    pos = jnp.zeros((64, 128), jnp.int32)
    blk = lax.broadcasted_iota(jnp.int32, (1, 128), 1)
    ebv = jnp.zeros((1, 128), jnp.int32)
    base = jnp.int32(0)
    for e in range(E):
        M = (R == e).astype(jnp.float32)
        lanepre = jnp.dot(M, SU, preferred_element_type=jnp.float32)
        tot = jnp.sum(M, axis=1, keepdims=True)
        rowpre = jnp.dot(SL, tot, preferred_element_type=jnp.float32)
        rank = (lanepre + rowpre).astype(jnp.int32)
        cnt = jnp.sum(M).astype(jnp.int32)
        cntpad = ((cnt + B - 1) // B) * B
        pos = jnp.where(R == e, base + rank, pos)
        base = base + cntpad
        ebv = ebv + (blk * B >= base).astype(jnp.int32)
    pos_ref[...] = pos
    eb_ref[...] = jnp.where(blk == 127, base // B, jnp.minimum(ebv, E - 1))


def _ffn_body(seb_ref, x_ref, gw_ref, uw_ref, dw_ref, y_ref):
    b = pl.program_id(0)
    nact = seb_ref[127]

    @pl.when(b < nact)
    def _():
        x = x_ref[...].astype(jnp.bfloat16)
        g = jnp.dot(x, gw_ref[0].T, preferred_element_type=jnp.float32)
        u = jnp.dot(x, uw_ref[0].T, preferred_element_type=jnp.float32)
        h = g * lax.logistic(g) * u
        y_ref[...] = jnp.dot(h.astype(jnp.bfloat16), dw_ref[0].T,
                             preferred_element_type=jnp.float32)


def _make_gather():
    mesh = plsc.VectorSubcoreMesh(core_axis_name="c", subcore_axis_name="s")

    @functools.partial(
        pl.kernel, mesh=mesh,
        out_type=jax.ShapeDtypeStruct((NPAD, HIDDEN), jnp.float32),
        scratch_types=[
            pltpu.VMEM((CH,), jnp.int32),
            pltpu.VMEM((CH,), jnp.int32),
            pltpu.VMEM((CH, HIDDEN), jnp.float32),
            pltpu.SemaphoreType.DMA,
        ],
    )
    def gather_k(x_hbm, tok_hbm, pos_hbm, xs_hbm, tok_v, pos_v, rows_v, sem):
        wid = lax.axis_index("s") * 2 + lax.axis_index("c")
        base = wid * PPW

        def chunk(c, carry):
            off = base + c * CH
            pltpu.sync_copy(tok_hbm.at[pl.ds(off, CH)], tok_v)
            pltpu.sync_copy(pos_hbm.at[pl.ds(off, CH)], pos_v)
            pltpu.async_copy(x_hbm.at[tok_v], rows_v, sem).wait()
            pltpu.async_copy(rows_v, xs_hbm.at[pos_v], sem).wait()
            return carry

        lax.fori_loop(0, PPW // CH, chunk, 0)

    return gather_k


def _make_combine():
    mesh = plsc.VectorSubcoreMesh(core_axis_name="c", subcore_axis_name="s")

    @functools.partial(
        pl.kernel, mesh=mesh,
        out_type=jax.ShapeDtypeStruct((T, HIDDEN), jnp.float32),
        scratch_types=[
            pltpu.VMEM((CC,), jnp.int32),
            pltpu.VMEM((CC,), jnp.int32),
            pltpu.VMEM((CC, HIDDEN), jnp.float32),
            pltpu.VMEM((CC, HIDDEN), jnp.float32),
            pltpu.VMEM((CC, 16), jnp.float32),
            pltpu.VMEM((CC, 16), jnp.float32),
            pltpu.VMEM((CC, HIDDEN), jnp.float32),
            pltpu.SemaphoreType.DMA,
        ],
    )
    def combine_k(y_hbm, p0_hbm, p1_hbm, w1_hbm, w2_hbm, out_hbm,
                  i0_v, i1_v, y0_v, y1_v, w1_v, w2_v, o_v, sem):
        wid = lax.axis_index("s") * 2 + lax.axis_index("c")
        base = wid * TPW

        def chunk(c, carry):
            off = base + c * CC
            pltpu.sync_copy(p0_hbm.at[pl.ds(off, CC)], i0_v)
            pltpu.sync_copy(p1_hbm.at[pl.ds(off, CC)], i1_v)
            pltpu.sync_copy(w1_hbm.at[pl.ds(off, CC)], w1_v)
            pltpu.sync_copy(w2_hbm.at[pl.ds(off, CC)], w2_v)
            cp0 = pltpu.async_copy(y_hbm.at[i0_v], y0_v, sem)
            cp1 = pltpu.async_copy(y_hbm.at[i1_v], y1_v, sem)
            cp0.wait()
            cp1.wait()

            def tok(j, carry2):
                wv1 = w1_v[j]
                wv2 = w2_v[j]
                for k in range(HIDDEN // 16):
                    sl = pl.ds(k * 16, 16)
                    o_v[j, sl] = wv1 * y0_v[j, sl] + wv2 * y1_v[j, sl]
                return carry2

            lax.fori_loop(0, CC, tok, 0)
            pltpu.sync_copy(o_v, out_hbm.at[pl.ds(off, CC)])
            return carry

        lax.fori_loop(0, TPW // CC, chunk, 0)

    return combine_k


@jax.jit
def kernel(x, router_w, router_b, gate_w, up_w, down_w):
    batch_shape = x.shape[:-1]
    xf = x.reshape(-1, HIDDEN)

    sel_out, w_out = pl.pallas_call(
        _router_body,
        grid=(T // BTR,),
        in_specs=[
            pl.BlockSpec((BTR, HIDDEN), lambda t: (t, 0)),
            pl.BlockSpec((E, HIDDEN), lambda t: (0, 0)),
            pl.BlockSpec((1, E), lambda t: (0, 0)),
        ],
        out_specs=[
            pl.BlockSpec((BTR, 128), lambda t: (t, 0)),
            pl.BlockSpec((BTR, 128), lambda t: (t, 0)),
        ],
        out_shape=[
            jax.ShapeDtypeStruct((T, 128), jnp.int32),
            jax.ShapeDtypeStruct((T, 128), jnp.float32),
        ],
    )(xf, router_w, router_b.reshape(1, E))

    pairs = sel_out[:, :2].reshape(64, 128)
    pos, eb = pl.pallas_call(
        _dispatch_body,
        in_specs=[pl.BlockSpec((64, 128), lambda: (0, 0))],
        out_specs=[
            pl.BlockSpec((64, 128), lambda: (0, 0)),
            pl.BlockSpec((1, 128), lambda: (0, 0)),
        ],
        out_shape=[
            jax.ShapeDtypeStruct((64, 128), jnp.int32),
            jax.ShapeDtypeStruct((1, 128), jnp.int32),
        ],
    )(pairs)

    pos_flat = pos.reshape(P)
    tok_flat = (jnp.arange(P, dtype=jnp.int32) // 2)
    x_sorted = _make_gather()(xf, tok_flat, pos_flat)

    seb = eb.reshape(128)
    grid_spec = pltpu.PrefetchScalarGridSpec(
        num_scalar_prefetch=1,
        grid=(NBMAX,),
        in_specs=[
            pl.BlockSpec((B, HIDDEN), lambda b, seb: (b, 0)),
            pl.BlockSpec((1, INTER, HIDDEN), lambda b, seb: (seb[b], 0, 0)),
            pl.BlockSpec((1, INTER, HIDDEN), lambda b, seb: (seb[b], 0, 0)),
            pl.BlockSpec((1, HIDDEN, INTER), lambda b, seb: (seb[b], 0, 0)),
        ],
        out_specs=pl.BlockSpec((B, HIDDEN), lambda b, seb: (b, 0)),
    )
    y_sorted = pl.pallas_call(
        _ffn_body,
        grid_spec=grid_spec,
        out_shape=jax.ShapeDtypeStruct((NPAD, HIDDEN), jnp.float32),
    )(seb, x_sorted, gate_w.astype(jnp.bfloat16), up_w.astype(jnp.bfloat16),
      down_w.astype(jnp.bfloat16))

    p0 = pos_flat[0::2]
    p1 = pos_flat[1::2]
    w1b = jnp.broadcast_to(w_out[:, 0:1], (T, 16))
    w2b = jnp.broadcast_to(w_out[:, 1:2], (T, 16))
    out = _make_combine()(y_sorted, p0, p1, w1b, w2b)

    return out.reshape(*batch_shape, HIDDEN)


# speedup vs baseline: 2.7864x; 1.0385x over previous
import functools

import jax
import jax.numpy as jnp
from jax import lax
from jax.experimental import pallas as pl
from jax.experimental.pallas import tpu as pltpu
from jax.experimental.pallas import tpu_sc as plsc

HIDDEN = 1024
INTER = 2816
E = 8

T = 4096
P = 2 * T
B = 256
NBMAX = P // B + E
NPAD = NBMAX * B
BTR = 512

NW = 32
PPW = P // NW
CH = 64
TPW = T // NW
CC = 32


def _router_body(x_ref, rw_ref, rb_ref, sel_ref, w_ref):
    x = x_ref[...]
    logits = jnp.dot(x, rw_ref[...].T, preferred_element_type=jnp.float32)
    logits = logits + rb_ref[...]
    m = jnp.max(logits, axis=-1, keepdims=True)
    ex = jnp.exp(logits - m)
    probs = ex / jnp.sum(ex, axis=-1, keepdims=True)

    lane = lax.broadcasted_iota(jnp.int32, (BTR, E), 1)
    m1 = jnp.max(probs, axis=-1, keepdims=True)
    a1 = jnp.min(jnp.where(probs == m1, lane, E), axis=-1, keepdims=True)
    probs2 = jnp.where(lane == a1, -1.0, probs)
    m2 = jnp.max(probs2, axis=-1, keepdims=True)
    a2 = jnp.min(jnp.where(probs2 == m2, lane, E), axis=-1, keepdims=True)

    denom = m1 + m2
    w1 = m1 / denom
    w2 = m2 / denom
    zi = jnp.zeros((BTR, 126), jnp.int32)
    zf = jnp.zeros((BTR, 126), jnp.float32)
    sel_ref[...] = jnp.concatenate([a1, a2, zi], axis=-1)
    w_ref[...] = jnp.concatenate([w1, w2, zf], axis=-1)


def _dispatch_body(pairs_ref, pos_ref, eb_ref):
    R = pairs_ref[...]
    r0 = lax.broadcasted_iota(jnp.int32, (128, 128), 0)
    r1 = lax.broadcasted_iota(jnp.int32, (128, 128), 1)
    SU = (r0 < r1).astype(jnp.float32)
    s0 = lax.broadcasted_iota(jnp.int32, (64, 64), 0)
    s1 = lax.broadcasted_iota(jnp.int32, (64, 64), 1)
    SL = (s1 < s0).astype(jnp.float32)

    pos = jnp.zeros((64, 128), jnp.int32)
    blk = lax.broadcasted_iota(jnp.int32, (1, 128), 1)
    ebv = jnp.zeros((1, 128), jnp.int32)
    base = jnp.int32(0)
    for e in range(E):
        M = (R == e).astype(jnp.float32)
        lanepre = jnp.dot(M, SU, preferred_element_type=jnp.float32)
        tot = jnp.sum(M, axis=1, keepdims=True)
        rowpre = jnp.dot(SL, tot, preferred_element_type=jnp.float32)
        rank = (lanepre + rowpre).astype(jnp.int32)
        cnt = jnp.sum(M).astype(jnp.int32)
        cntpad = ((cnt + B - 1) // B) * B
        pos = jnp.where(R == e, base + rank, pos)
        base = base + cntpad
        ebv = ebv + (blk * B >= base).astype(jnp.int32)
    pos_ref[...] = pos
    eb_ref[...] = jnp.where(blk == 127, base // B, jnp.minimum(ebv, E - 1))


IBLK = 1408
NI = INTER // IBLK


def _gateup_body(seb_ref, x_ref, gw_ref, uw_ref, h_ref):
    b = pl.program_id(1)
    nact = seb_ref[127]

    @pl.when(b < nact)
    def _():
        x = x_ref[...]
        g = jnp.dot(x, gw_ref[0].T, preferred_element_type=jnp.float32)
        u = jnp.dot(x, uw_ref[0].T, preferred_element_type=jnp.float32)
        h = g * lax.logistic(g) * u
        h_ref[...] = h.astype(jnp.bfloat16)


def _down_body(seb_ref, h_ref, dw_ref, y_ref):
    b = pl.program_id(0)
    nact = seb_ref[127]

    @pl.when(b < nact)
    def _():
        h = h_ref[...].astype(jnp.float32)
        y_ref[...] = jnp.dot(h, dw_ref[0].T,
                             preferred_element_type=jnp.float32)


def _make_gather():
    mesh = plsc.VectorSubcoreMesh(core_axis_name="c", subcore_axis_name="s")

    @functools.partial(
        pl.kernel, mesh=mesh,
        out_type=jax.ShapeDtypeStruct((NPAD, HIDDEN), jnp.float32),
        scratch_types=[
            pltpu.VMEM((CH,), jnp.int32),
            pltpu.VMEM((CH,), jnp.int32),
            pltpu.VMEM((CH, HIDDEN), jnp.float32),
            pltpu.SemaphoreType.DMA,
        ],
    )
    def gather_k(x_hbm, tok_hbm, pos_hbm, xs_hbm, tok_v, pos_v, rows_v, sem):
        wid = lax.axis_index("s") * 2 + lax.axis_index("c")
        base = wid * PPW

        def chunk(c, carry):
            off = base + c * CH
            pltpu.sync_copy(tok_hbm.at[pl.ds(off, CH)], tok_v)
            pltpu.sync_copy(pos_hbm.at[pl.ds(off, CH)], pos_v)
            pltpu.async_copy(x_hbm.at[tok_v], rows_v, sem).wait()
            pltpu.async_copy(rows_v, xs_hbm.at[pos_v], sem).wait()
            return carry

        lax.fori_loop(0, PPW // CH, chunk, 0)

    return gather_k


def _make_combine():
    mesh = plsc.VectorSubcoreMesh(core_axis_name="c", subcore_axis_name="s")

    @functools.partial(
        pl.kernel, mesh=mesh,
        out_type=jax.ShapeDtypeStruct((T, HIDDEN), jnp.float32),
        scratch_types=[
            pltpu.VMEM((CC,), jnp.int32),
            pltpu.VMEM((CC,), jnp.int32),
            pltpu.VMEM((CC, HIDDEN), jnp.float32),
            pltpu.VMEM((CC, HIDDEN), jnp.float32),
            pltpu.VMEM((CC, 16), jnp.float32),
            pltpu.VMEM((CC, 16), jnp.float32),
            pltpu.VMEM((CC, HIDDEN), jnp.float32),
            pltpu.SemaphoreType.DMA,
        ],
    )
    def combine_k(y_hbm, p0_hbm, p1_hbm, w1_hbm, w2_hbm, out_hbm,
                  i0_v, i1_v, y0_v, y1_v, w1_v, w2_v, o_v, sem):
        wid = lax.axis_index("s") * 2 + lax.axis_index("c")
        base = wid * TPW

        def chunk(c, carry):
            off = base + c * CC
            pltpu.sync_copy(p0_hbm.at[pl.ds(off, CC)], i0_v)
            pltpu.sync_copy(p1_hbm.at[pl.ds(off, CC)], i1_v)
            pltpu.sync_copy(w1_hbm.at[pl.ds(off, CC)], w1_v)
            pltpu.sync_copy(w2_hbm.at[pl.ds(off, CC)], w2_v)
            cp0 = pltpu.async_copy(y_hbm.at[i0_v], y0_v, sem)
            cp1 = pltpu.async_copy(y_hbm.at[i1_v], y1_v, sem)
            cp0.wait()
            cp1.wait()

            def tok(j, carry2):
                wv1 = w1_v[j]
                wv2 = w2_v[j]
                for k in range(HIDDEN // 16):
                    sl = pl.ds(k * 16, 16)
                    o_v[j, sl] = wv1 * y0_v[j, sl] + wv2 * y1_v[j, sl]
                return carry2

            lax.fori_loop(0, CC, tok, 0)
            pltpu.sync_copy(o_v, out_hbm.at[pl.ds(off, CC)])
            return carry

        lax.fori_loop(0, TPW // CC, chunk, 0)

    return combine_k


@jax.jit
def kernel(x, router_w, router_b, gate_w, up_w, down_w):
    batch_shape = x.shape[:-1]
    xf = x.reshape(-1, HIDDEN)

    sel_out, w_out = pl.pallas_call(
        _router_body,
        grid=(T // BTR,),
        in_specs=[
            pl.BlockSpec((BTR, HIDDEN), lambda t: (t, 0)),
            pl.BlockSpec((E, HIDDEN), lambda t: (0, 0)),
            pl.BlockSpec((1, E), lambda t: (0, 0)),
        ],
        out_specs=[
            pl.BlockSpec((BTR, 128), lambda t: (t, 0)),
            pl.BlockSpec((BTR, 128), lambda t: (t, 0)),
        ],
        out_shape=[
            jax.ShapeDtypeStruct((T, 128), jnp.int32),
            jax.ShapeDtypeStruct((T, 128), jnp.float32),
        ],
    )(xf, router_w, router_b.reshape(1, E))

    pairs = sel_out[:, :2].reshape(64, 128)
    pos, eb = pl.pallas_call(
        _dispatch_body,
        in_specs=[pl.BlockSpec((64, 128), lambda: (0, 0))],
        out_specs=[
            pl.BlockSpec((64, 128), lambda: (0, 0)),
            pl.BlockSpec((1, 128), lambda: (0, 0)),
        ],
        out_shape=[
            jax.ShapeDtypeStruct((64, 128), jnp.int32),
            jax.ShapeDtypeStruct((1, 128), jnp.int32),
        ],
    )(pairs)

    pos_flat = pos.reshape(P)
    tok_flat = (jnp.arange(P, dtype=jnp.int32) // 2)
    x_sorted = _make_gather()(xf, tok_flat, pos_flat)

    seb = eb.reshape(128)
    h_sorted = pl.pallas_call(
        _gateup_body,
        grid_spec=pltpu.PrefetchScalarGridSpec(
            num_scalar_prefetch=1,
            grid=(NI, NBMAX),
            in_specs=[
                pl.BlockSpec((B, HIDDEN), lambda i, b, seb: (b, 0)),
                pl.BlockSpec((1, IBLK, HIDDEN),
                             lambda i, b, seb: (seb[b], i, 0)),
                pl.BlockSpec((1, IBLK, HIDDEN),
                             lambda i, b, seb: (seb[b], i, 0)),
            ],
            out_specs=pl.BlockSpec((B, IBLK), lambda i, b, seb: (b, i)),
        ),
        out_shape=jax.ShapeDtypeStruct((NPAD, INTER), jnp.bfloat16),
    )(seb, x_sorted, gate_w, up_w)

    y_sorted = pl.pallas_call(
        _down_body,
        grid_spec=pltpu.PrefetchScalarGridSpec(
            num_scalar_prefetch=1,
            grid=(NBMAX,),
            in_specs=[
                pl.BlockSpec((B, INTER), lambda b, seb: (b, 0)),
                pl.BlockSpec((1, HIDDEN, INTER), lambda b, seb: (seb[b], 0, 0)),
            ],
            out_specs=pl.BlockSpec((B, HIDDEN), lambda b, seb: (b, 0)),
        ),
        out_shape=jax.ShapeDtypeStruct((NPAD, HIDDEN), jnp.float32),
    )(seb, h_sorted, down_w)

    p0 = pos_flat[0::2]
    p1 = pos_flat[1::2]
    w1b = jnp.broadcast_to(w_out[:, 0:1], (T, 16))
    w2b = jnp.broadcast_to(w_out[:, 1:2], (T, 16))
    out = _make_combine()(y_sorted, p0, p1, w1b, w2b)

    return out.reshape(*batch_shape, HIDDEN)
